# Initial kernel scaffold; baseline (speedup 1.0000x reference)
import jax
import jax.numpy as jnp
from jax.experimental import pallas as pl


def kernel(x, v, indices_in, indices_out):
    raise NotImplementedError("write your pallas kernel here")


# speedup vs baseline: 2.4486x; 2.4486x over previous
import dataclasses

import jax
import jax.numpy as jnp
from jax import lax
from jax.experimental import pallas as pl
from jax.experimental.pallas import tpu as pltpu
from jax.experimental.pallas import tpu_sc as plsc

OUT_SIZE = 16384
NCORES = 2
NSUB = 16
LANES = 16
EBLK = 128
BROWS = 8
WCHUNK = 64
NCHUNK = 4


def _sc_spmm(x4, ii2d, io2d, v1d, *, rows_per_tile, in_size):
    mesh = plsc.VectorSubcoreMesh(core_axis_name="c", subcore_axis_name="s")
    out_type = jax.ShapeDtypeStruct((NCHUNK * OUT_SIZE, WCHUNK), jnp.float32)
    rows_out = OUT_SIZE // NSUB

    nblocks = rows_per_tile // BROWS

    def body(x_hbm, ii_hbm, io_hbm, v_hbm, out_hbm,
             acc, ii_v, io_v, gi_v, v_v, rows_b, sem):
        c = lax.axis_index("c")
        s = lax.axis_index("s")
        row0 = s * rows_per_tile

        zero16 = jnp.zeros((LANES,), jnp.float32)

        for ci in range(NCHUNK // NCORES):
            chunk = c * (NCHUNK // NCORES) + ci

            @pl.loop(0, EBLK)
            def _(r):
                for k in range(WCHUNK // LANES):
                    rows_b[r, pl.ds(k * LANES, LANES)] = zero16

            @pl.loop(0, rows_out // EBLK)
            def _(k):
                pltpu.sync_copy(rows_b, acc.at[pl.ds(s * rows_out + k * EBLK, EBLK)])
            plsc.subcore_barrier()

            off16 = jnp.full((LANES,), chunk * in_size, jnp.int32)

            @pl.loop(0, nblocks)
            def _(nb):
                brow = row0 + nb * BROWS
                pltpu.sync_copy(ii_hbm.at[pl.ds(brow, BROWS)], ii_v)
                pltpu.sync_copy(io_hbm.at[pl.ds(brow, BROWS)], io_v)
                pltpu.sync_copy(v_hbm.at[pl.ds(brow * EBLK, BROWS * EBLK)], v_v)

                @pl.loop(0, BROWS)
                def _(r):
                    for k in range(EBLK // LANES):
                        sl = pl.ds(k * LANES, LANES)
                        gi_v[r, sl] = ii_v[r, sl] + off16

                @pl.loop(0, BROWS)
                def _(j):
                    pltpu.async_copy(x_hbm.at[gi_v.at[j]], rows_b, sem).wait()

                    @pl.loop(0, EBLK)
                    def _(e):
                        vspl = plsc.load_gather(
                            v_v, [jnp.full((LANES,), j * EBLK + e, jnp.int32)])
                        for k in range(WCHUNK // LANES):
                            sl = pl.ds(k * LANES, LANES)
                            rows_b[e, sl] = rows_b[e, sl] * vspl

                    pltpu.sync_copy(rows_b, acc.at[io_v.at[j]], add=True)

            plsc.subcore_barrier()
            pltpu.sync_copy(
                acc.at[pl.ds(s * rows_out, rows_out)],
                out_hbm.at[pl.ds(chunk * OUT_SIZE + s * rows_out, rows_out)])
            plsc.subcore_barrier()

    cp = pltpu.CompilerParams()
    if "needs_layout_passes" in pltpu.CompilerParams.__dataclass_fields__:
        cp = dataclasses.replace(cp, needs_layout_passes=False)
    if "use_tc_tiling_on_sc" in pltpu.CompilerParams.__dataclass_fields__:
        cp = dataclasses.replace(cp, use_tc_tiling_on_sc=False)
    run = pl.kernel(
        body,
        out_type=out_type,
        mesh=mesh,
        compiler_params=cp,
        scratch_types=[
            pltpu.VMEM_SHARED((OUT_SIZE, WCHUNK), jnp.float32),
            pltpu.VMEM((BROWS, EBLK), jnp.int32),
            pltpu.VMEM((BROWS, EBLK), jnp.int32),
            pltpu.VMEM((BROWS, EBLK), jnp.int32),
            pltpu.VMEM((BROWS * EBLK,), jnp.float32),
            pltpu.VMEM((EBLK, WCHUNK), jnp.float32),
            pltpu.SemaphoreType.DMA,
        ],
    )
    return run(x4, ii2d, io2d, v1d)


@jax.jit
def kernel(x, v, indices_in, indices_out):
    batch, in_size = x.shape
    nnz = v.shape[0]
    assert batch == NCHUNK * WCHUNK

    per_tile = -(-nnz // (NSUB * EBLK * 8)) * EBLK * 8
    nnz_pad = per_tile * NSUB
    pad = nnz_pad - nnz
    ii = jnp.concatenate([indices_in, jnp.zeros((pad,), jnp.int32)])
    io = jnp.concatenate([indices_out, jnp.zeros((pad,), jnp.int32)])
    vp = jnp.concatenate([v, jnp.zeros((pad,), jnp.float32)])
    ii2d = ii.reshape(nnz_pad // EBLK, EBLK)
    io2d = io.reshape(nnz_pad // EBLK, EBLK)

    x4 = (x.T.reshape(in_size, NCHUNK, WCHUNK)
          .transpose(1, 0, 2).reshape(NCHUNK * in_size, WCHUNK))

    yt4 = _sc_spmm(x4, ii2d, io2d, vp,
                   rows_per_tile=per_tile // EBLK, in_size=in_size)

    y = (yt4.reshape(NCHUNK, OUT_SIZE, WCHUNK)
         .transpose(1, 0, 2).reshape(OUT_SIZE, batch).T)
    return y


# speedup vs baseline: 3.7135x; 1.5166x over previous
import dataclasses

import jax
import jax.numpy as jnp
from jax import lax
from jax.experimental import pallas as pl
from jax.experimental.pallas import tpu as pltpu
from jax.experimental.pallas import tpu_sc as plsc

OUT_SIZE = 16384
NCORES = 2
NSUB = 16
LANES = 16
EBLK = 128
SROWS = 72
NRING = 4
WCHUNK = 64
NCHUNK = 4


def _sc_spmm(x4, ii2d, io2d, v1d, *, rows_per_tile, in_size):
    mesh = plsc.VectorSubcoreMesh(core_axis_name="c", subcore_axis_name="s")
    out_type = jax.ShapeDtypeStruct((NCHUNK * OUT_SIZE, WCHUNK), jnp.float32)
    rows_out = OUT_SIZE // NSUB

    stages = []
    r = 0
    while r < rows_per_tile:
        n = min(SROWS, rows_per_tile - r)
        stages.append((r, n))
        r += n

    def body(x_hbm, ii_hbm, io_hbm, v_hbm, out_hbm,
             acc, ii_v, io_v, v_v, rb0, rb1, rb2, rb3, gsem, ssem):
        c = lax.axis_index("c")
        s = lax.axis_index("s")
        row0 = s * rows_per_tile
        rbufs = [rb0, rb1, rb2, rb3]

        zero16 = jnp.zeros((LANES,), jnp.float32)

        def mul_block(rb, j):
            @pl.loop(0, EBLK)
            def _(e):
                vspl = plsc.load_gather(
                    v_v, [jnp.full((LANES,), j * EBLK + e, jnp.int32)])
                for k in range(WCHUNK // LANES):
                    sl = pl.ds(k * LANES, LANES)
                    rb[e, sl] = rb[e, sl] * vspl

        for ci in range(NCHUNK // NCORES):
            chunk = c * (NCHUNK // NCORES) + ci

            @pl.loop(0, EBLK)
            def _(r):
                for k in range(WCHUNK // LANES):
                    rb0[r, pl.ds(k * LANES, LANES)] = zero16

            @pl.loop(0, rows_out // EBLK)
            def _(k):
                pltpu.sync_copy(rb0, acc.at[pl.ds(s * rows_out + k * EBLK, EBLK)])
            plsc.subcore_barrier()

            off16 = jnp.full((LANES,), chunk * in_size, jnp.int32)

            for srow, slen in stages:
                brow = row0 + srow
                pltpu.sync_copy(ii_hbm.at[pl.ds(brow, slen)],
                                ii_v.at[pl.ds(0, slen)])
                pltpu.sync_copy(io_hbm.at[pl.ds(brow, slen)],
                                io_v.at[pl.ds(0, slen)])
                pltpu.sync_copy(v_hbm.at[pl.ds(brow * EBLK, slen * EBLK)],
                                v_v.at[pl.ds(0, slen * EBLK)])

                @pl.loop(0, slen)
                def _(r):
                    for k in range(EBLK // LANES):
                        sl = pl.ds(k * LANES, LANES)
                        ii_v[r, sl] = ii_v[r, sl] + off16

                for q in range(min(NRING - 1, slen)):
                    pltpu.async_copy(x_hbm.at[ii_v.at[q]], rbufs[q], gsem)

                @pl.loop(0, slen // NRING)
                def _(p):
                    for q in range(NRING):
                        j = p * NRING + q
                        rb = rbufs[q]
                        pltpu.make_async_copy(
                            x_hbm.at[ii_v.at[j]], rb, gsem).wait()
                        mul_block(rb, j)
                        pltpu.async_copy(rb, acc.at[io_v.at[j]], ssem, add=True)

                        @pl.when(j >= 1)
                        def _():
                            jm = j - 1
                            pltpu.make_async_copy(
                                rbufs[(q + NRING - 1) % NRING],
                                acc.at[io_v.at[jm]], ssem).wait()

                        @pl.when(j + NRING - 1 < slen)
                        def _():
                            jn = j + NRING - 1
                            pltpu.async_copy(
                                x_hbm.at[ii_v.at[jn]],
                                rbufs[(q + NRING - 1) % NRING], gsem)

                pltpu.make_async_copy(
                    rbufs[(slen - 1) % NRING],
                    acc.at[io_v.at[slen - 1]], ssem).wait()

            plsc.subcore_barrier()
            pltpu.sync_copy(
                acc.at[pl.ds(s * rows_out, rows_out)],
                out_hbm.at[pl.ds(chunk * OUT_SIZE + s * rows_out, rows_out)])
            plsc.subcore_barrier()

    cp = pltpu.CompilerParams()
    if "needs_layout_passes" in pltpu.CompilerParams.__dataclass_fields__:
        cp = dataclasses.replace(cp, needs_layout_passes=False)
    if "use_tc_tiling_on_sc" in pltpu.CompilerParams.__dataclass_fields__:
        cp = dataclasses.replace(cp, use_tc_tiling_on_sc=False)
    run = pl.kernel(
        body,
        out_type=out_type,
        mesh=mesh,
        compiler_params=cp,
        scratch_types=[
            pltpu.VMEM_SHARED((OUT_SIZE, WCHUNK), jnp.float32),
            pltpu.VMEM((SROWS, EBLK), jnp.int32),
            pltpu.VMEM((SROWS, EBLK), jnp.int32),
            pltpu.VMEM((SROWS * EBLK,), jnp.float32),
            pltpu.VMEM((EBLK, WCHUNK), jnp.float32),
            pltpu.VMEM((EBLK, WCHUNK), jnp.float32),
            pltpu.VMEM((EBLK, WCHUNK), jnp.float32),
            pltpu.VMEM((EBLK, WCHUNK), jnp.float32),
            pltpu.SemaphoreType.DMA,
            pltpu.SemaphoreType.DMA,
        ],
    )
    return run(x4, ii2d, io2d, v1d)


@jax.jit
def kernel(x, v, indices_in, indices_out):
    batch, in_size = x.shape
    nnz = v.shape[0]
    assert batch == NCHUNK * WCHUNK

    per_tile = -(-nnz // (NSUB * EBLK * 8)) * EBLK * 8
    nnz_pad = per_tile * NSUB
    pad = nnz_pad - nnz
    ii = jnp.concatenate([indices_in, jnp.zeros((pad,), jnp.int32)])
    io = jnp.concatenate([indices_out, jnp.zeros((pad,), jnp.int32)])
    vp = jnp.concatenate([v, jnp.zeros((pad,), jnp.float32)])
    ii2d = ii.reshape(nnz_pad // EBLK, EBLK)
    io2d = io.reshape(nnz_pad // EBLK, EBLK)

    x4 = (x.T.reshape(in_size, NCHUNK, WCHUNK)
          .transpose(1, 0, 2).reshape(NCHUNK * in_size, WCHUNK))

    yt4 = _sc_spmm(x4, ii2d, io2d, vp,
                   rows_per_tile=per_tile // EBLK, in_size=in_size)

    y = (yt4.reshape(NCHUNK, OUT_SIZE, WCHUNK)
         .transpose(1, 0, 2).reshape(OUT_SIZE, batch).T)
    return y


# speedup vs baseline: 3.9361x; 1.0599x over previous
import dataclasses

import jax
import jax.numpy as jnp
from jax import lax
from jax.experimental import pallas as pl
from jax.experimental.pallas import tpu as pltpu
from jax.experimental.pallas import tpu_sc as plsc

OUT_SIZE = 16384
NCORES = 2
NSUB = 16
LANES = 16
EBLK = 128
SROWS = 72
NRING = 4
WCHUNK = 64
NCHUNK = 4


def _sc_spmm(x4, ii2d, io2d, v1d, *, rows_per_tile, in_size):
    mesh = plsc.VectorSubcoreMesh(core_axis_name="c", subcore_axis_name="s")
    out_type = jax.ShapeDtypeStruct((NCHUNK * OUT_SIZE, WCHUNK), jnp.float32)
    rows_out = OUT_SIZE // NSUB

    stages = []
    r = 0
    while r < rows_per_tile:
        n = min(SROWS, rows_per_tile - r)
        stages.append((r, n))
        r += n

    def body(x_hbm, ii_hbm, io_hbm, v_hbm, out_hbm,
             acc, ii_v, io_v, v_v, rb0, rb1, rb2, rb3, gsem, ssem):
        c = lax.axis_index("c")
        s = lax.axis_index("s")
        row0 = s * rows_per_tile
        rbufs = [rb0, rb1, rb2, rb3]

        zero16 = jnp.zeros((LANES,), jnp.float32)

        def mul_block(rb, j):
            @plsc.parallel_loop(0, EBLK, unroll=4)
            def _(e):
                vspl = plsc.load_gather(
                    v_v, [jnp.full((LANES,), j * EBLK + e, jnp.int32)])
                for k in range(WCHUNK // LANES):
                    sl = pl.ds(k * LANES, LANES)
                    rb[e, sl] = rb[e, sl] * vspl

        for ci in range(NCHUNK // NCORES):
            chunk = c * (NCHUNK // NCORES) + ci

            @pl.loop(0, EBLK)
            def _(r):
                for k in range(WCHUNK // LANES):
                    rb0[r, pl.ds(k * LANES, LANES)] = zero16

            @pl.loop(0, rows_out // EBLK)
            def _(k):
                pltpu.sync_copy(rb0, acc.at[pl.ds(s * rows_out + k * EBLK, EBLK)])
            plsc.subcore_barrier()

            off16 = jnp.full((LANES,), chunk * in_size, jnp.int32)

            for srow, slen in stages:
                brow = row0 + srow
                pltpu.sync_copy(ii_hbm.at[pl.ds(brow, slen)],
                                ii_v.at[pl.ds(0, slen)])
                pltpu.sync_copy(io_hbm.at[pl.ds(brow, slen)],
                                io_v.at[pl.ds(0, slen)])
                pltpu.sync_copy(v_hbm.at[pl.ds(brow * EBLK, slen * EBLK)],
                                v_v.at[pl.ds(0, slen * EBLK)])

                @pl.loop(0, slen)
                def _(r):
                    for k in range(EBLK // LANES):
                        sl = pl.ds(k * LANES, LANES)
                        ii_v[r, sl] = ii_v[r, sl] + off16

                for q in range(min(NRING - 1, slen)):
                    pltpu.async_copy(x_hbm.at[ii_v.at[q]], rbufs[q], gsem)

                @pl.loop(0, slen // NRING)
                def _(p):
                    for q in range(NRING):
                        j = p * NRING + q
                        rb = rbufs[q]
                        pltpu.make_async_copy(
                            x_hbm.at[ii_v.at[j]], rb, gsem).wait()
                        mul_block(rb, j)
                        pltpu.async_copy(rb, acc.at[io_v.at[j]], ssem, add=True)

                        @pl.when(j >= 1)
                        def _():
                            jm = j - 1
                            pltpu.make_async_copy(
                                rbufs[(q + NRING - 1) % NRING],
                                acc.at[io_v.at[jm]], ssem).wait()

                        @pl.when(j + NRING - 1 < slen)
                        def _():
                            jn = j + NRING - 1
                            pltpu.async_copy(
                                x_hbm.at[ii_v.at[jn]],
                                rbufs[(q + NRING - 1) % NRING], gsem)

                pltpu.make_async_copy(
                    rbufs[(slen - 1) % NRING],
                    acc.at[io_v.at[slen - 1]], ssem).wait()

            plsc.subcore_barrier()
            pltpu.sync_copy(
                acc.at[pl.ds(s * rows_out, rows_out)],
                out_hbm.at[pl.ds(chunk * OUT_SIZE + s * rows_out, rows_out)])
            plsc.subcore_barrier()

    cp = pltpu.CompilerParams()
    if "needs_layout_passes" in pltpu.CompilerParams.__dataclass_fields__:
        cp = dataclasses.replace(cp, needs_layout_passes=False)
    if "use_tc_tiling_on_sc" in pltpu.CompilerParams.__dataclass_fields__:
        cp = dataclasses.replace(cp, use_tc_tiling_on_sc=False)
    run = pl.kernel(
        body,
        out_type=out_type,
        mesh=mesh,
        compiler_params=cp,
        scratch_types=[
            pltpu.VMEM_SHARED((OUT_SIZE, WCHUNK), jnp.float32),
            pltpu.VMEM((SROWS, EBLK), jnp.int32),
            pltpu.VMEM((SROWS, EBLK), jnp.int32),
            pltpu.VMEM((SROWS * EBLK,), jnp.float32),
            pltpu.VMEM((EBLK, WCHUNK), jnp.float32),
            pltpu.VMEM((EBLK, WCHUNK), jnp.float32),
            pltpu.VMEM((EBLK, WCHUNK), jnp.float32),
            pltpu.VMEM((EBLK, WCHUNK), jnp.float32),
            pltpu.SemaphoreType.DMA,
            pltpu.SemaphoreType.DMA,
        ],
    )
    return run(x4, ii2d, io2d, v1d)


@jax.jit
def kernel(x, v, indices_in, indices_out):
    batch, in_size = x.shape
    nnz = v.shape[0]
    assert batch == NCHUNK * WCHUNK

    per_tile = -(-nnz // (NSUB * EBLK * 8)) * EBLK * 8
    nnz_pad = per_tile * NSUB
    pad = nnz_pad - nnz
    ii = jnp.concatenate([indices_in, jnp.zeros((pad,), jnp.int32)])
    io = jnp.concatenate([indices_out, jnp.zeros((pad,), jnp.int32)])
    vp = jnp.concatenate([v, jnp.zeros((pad,), jnp.float32)])
    ii2d = ii.reshape(nnz_pad // EBLK, EBLK)
    io2d = io.reshape(nnz_pad // EBLK, EBLK)

    x4 = (x.T.reshape(in_size, NCHUNK, WCHUNK)
          .transpose(1, 0, 2).reshape(NCHUNK * in_size, WCHUNK))

    yt4 = _sc_spmm(x4, ii2d, io2d, vp,
                   rows_per_tile=per_tile // EBLK, in_size=in_size)

    y = (yt4.reshape(NCHUNK, OUT_SIZE, WCHUNK)
         .transpose(1, 0, 2).reshape(OUT_SIZE, batch).T)
    return y


# speedup vs baseline: 5.2092x; 1.3234x over previous
import dataclasses

import jax
import jax.numpy as jnp
from jax import lax
from jax.experimental import pallas as pl
from jax.experimental.pallas import tpu as pltpu
from jax.experimental.pallas import tpu_sc as plsc

OUT_SIZE = 16384
NCORES = 2
NSUB = 16
LANES = 16
EBLK = 128
SROWS = 72
NRING = 4
WCHUNK = 64
NCHUNK = 4


def _sc_spmm(x4, ii2d, io2d, v1d, *, rows_per_tile, in_size):
    mesh = plsc.VectorSubcoreMesh(core_axis_name="c", subcore_axis_name="s")
    out_type = jax.ShapeDtypeStruct((NCHUNK * OUT_SIZE, WCHUNK), jnp.float32)
    rows_out = OUT_SIZE // NSUB

    stages = []
    r = 0
    while r < rows_per_tile:
        n = min(SROWS, rows_per_tile - r)
        assert n % NRING == 0 and n % 8 == 0
        stages.append((r, n))
        r += n

    def body(x_hbm, ii_hbm, io_hbm, v_hbm, out_hbm,
             acc, ii_v, io_v, v_v, rb0, rb1, rb2, rb3, sb0, sb1, gsem, ssem):
        c = lax.axis_index("c")
        s = lax.axis_index("s")
        row0 = s * rows_per_tile
        rbufs = [rb0, rb1, rb2, rb3]
        sbufs = [sb0, sb1]

        zero16 = jnp.zeros((LANES,), jnp.float32)

        def mul_block(rb, sb, j):
            @plsc.parallel_loop(0, EBLK, unroll=4)
            def _(e):
                vspl = plsc.load_gather(
                    v_v, [jnp.full((LANES,), j * EBLK + e, jnp.int32)])
                for g in range(WCHUNK // (2 * LANES)):
                    ab = rb[e, pl.ds(g * 2 * LANES, 2 * LANES)]
                    a, b = plsc.unpack(ab, format=plsc.PackFormat.INTERLEAVED)
                    sb[e, pl.ds(g * 2 * LANES, LANES)] = a * vspl
                    sb[e, pl.ds(g * 2 * LANES + LANES, LANES)] = b * vspl

        for ci in range(NCHUNK // NCORES):
            chunk = c * (NCHUNK // NCORES) + ci

            @pl.loop(0, EBLK)
            def _(r):
                for k in range(WCHUNK // LANES):
                    sb0[r, pl.ds(k * LANES, LANES)] = zero16

            @pl.loop(0, rows_out // EBLK)
            def _(k):
                pltpu.sync_copy(sb0, acc.at[pl.ds(s * rows_out + k * EBLK, EBLK)])
            plsc.subcore_barrier()

            off16 = jnp.full((LANES,), chunk * in_size, jnp.int32)

            for srow, slen in stages:
                brow = row0 + srow
                pltpu.sync_copy(ii_hbm.at[pl.ds(brow, slen)],
                                ii_v.at[pl.ds(0, slen)])
                pltpu.sync_copy(io_hbm.at[pl.ds(brow, slen)],
                                io_v.at[pl.ds(0, slen)])
                pltpu.sync_copy(v_hbm.at[pl.ds(brow * EBLK, slen * EBLK)],
                                v_v.at[pl.ds(0, slen * EBLK)])

                @pl.loop(0, slen)
                def _(r):
                    for k in range(EBLK // LANES):
                        sl = pl.ds(k * LANES, LANES)
                        ii_v[r, sl] = ii_v[r, sl] + off16

                for q in range(NRING - 1):
                    pltpu.async_copy(x_hbm.at[ii_v.at[q]], rbufs[q], gsem)

                @pl.loop(0, slen // NRING)
                def _(p):
                    for q in range(NRING):
                        j = p * NRING + q
                        rb = rbufs[q]
                        sb = sbufs[q % 2]
                        pltpu.make_async_copy(
                            x_hbm.at[ii_v.at[j]], rb, gsem).wait()

                        @pl.when(j + NRING - 1 < slen)
                        def _():
                            jn = j + NRING - 1
                            pltpu.async_copy(
                                x_hbm.at[ii_v.at[jn]],
                                rbufs[(q + NRING - 1) % NRING], gsem)

                        @pl.when(j >= 2)
                        def _():
                            jm = j - 2
                            pltpu.make_async_copy(
                                sbufs[q % 2], acc.at[io_v.at[jm]], ssem).wait()

                        mul_block(rb, sb, j)
                        pltpu.async_copy(sb, acc.at[io_v.at[j]], ssem, add=True)

                pltpu.make_async_copy(
                    sbufs[(slen - 2) % 2], acc.at[io_v.at[slen - 2]],
                    ssem).wait()
                pltpu.make_async_copy(
                    sbufs[(slen - 1) % 2], acc.at[io_v.at[slen - 1]],
                    ssem).wait()

            plsc.subcore_barrier()
            pltpu.sync_copy(
                acc.at[pl.ds(s * rows_out, rows_out)],
                out_hbm.at[pl.ds(chunk * OUT_SIZE + s * rows_out, rows_out)])
            plsc.subcore_barrier()

    cp = pltpu.CompilerParams()
    if "needs_layout_passes" in pltpu.CompilerParams.__dataclass_fields__:
        cp = dataclasses.replace(cp, needs_layout_passes=False)
    if "use_tc_tiling_on_sc" in pltpu.CompilerParams.__dataclass_fields__:
        cp = dataclasses.replace(cp, use_tc_tiling_on_sc=False)
    run = pl.kernel(
        body,
        out_type=out_type,
        mesh=mesh,
        compiler_params=cp,
        scratch_types=[
            pltpu.VMEM_SHARED((OUT_SIZE, WCHUNK), jnp.float32),
            pltpu.VMEM((SROWS, EBLK), jnp.int32),
            pltpu.VMEM((SROWS, EBLK), jnp.int32),
            pltpu.VMEM((SROWS * EBLK,), jnp.float32),
            pltpu.VMEM((EBLK, WCHUNK), jnp.bfloat16),
            pltpu.VMEM((EBLK, WCHUNK), jnp.bfloat16),
            pltpu.VMEM((EBLK, WCHUNK), jnp.bfloat16),
            pltpu.VMEM((EBLK, WCHUNK), jnp.bfloat16),
            pltpu.VMEM((EBLK, WCHUNK), jnp.float32),
            pltpu.VMEM((EBLK, WCHUNK), jnp.float32),
            pltpu.SemaphoreType.DMA,
            pltpu.SemaphoreType.DMA,
        ],
    )
    return run(x4, ii2d, io2d, v1d)


@jax.jit
def kernel(x, v, indices_in, indices_out):
    batch, in_size = x.shape
    nnz = v.shape[0]
    assert batch == NCHUNK * WCHUNK

    per_tile = -(-nnz // (NSUB * EBLK * 8)) * EBLK * 8
    nnz_pad = per_tile * NSUB
    pad = nnz_pad - nnz
    ii = jnp.concatenate([indices_in, jnp.zeros((pad,), jnp.int32)])
    io = jnp.concatenate([indices_out, jnp.zeros((pad,), jnp.int32)])
    vp = jnp.concatenate([v, jnp.zeros((pad,), jnp.float32)])
    ii2d = ii.reshape(nnz_pad // EBLK, EBLK)
    io2d = io.reshape(nnz_pad // EBLK, EBLK)

    x4 = (x.T.reshape(in_size, NCHUNK, WCHUNK)
          .transpose(1, 0, 2).reshape(NCHUNK * in_size, WCHUNK))
    x4 = (x4.reshape(-1, WCHUNK // (2 * LANES), 2, LANES)
          .transpose(0, 1, 3, 2).reshape(-1, WCHUNK))
    x4 = x4.astype(jnp.bfloat16)

    yt4 = _sc_spmm(x4, ii2d, io2d, vp,
                   rows_per_tile=per_tile // EBLK, in_size=in_size)

    y = (yt4.reshape(NCHUNK, OUT_SIZE, WCHUNK)
         .transpose(1, 0, 2).reshape(OUT_SIZE, batch).T)
    return y


# speedup vs baseline: 8.5865x; 1.6483x over previous
import dataclasses

import jax
import jax.numpy as jnp
from jax import lax
from jax.experimental import pallas as pl
from jax.experimental.pallas import tpu as pltpu
from jax.experimental.pallas import tpu_sc as plsc

OUT_SIZE = 16384
NCORES = 2
NSUB = 16
LANES = 16
EBLK = 128
SROWS = 72
NRING = 4
WCHUNK = 64
NCHUNK = 4


def _sc_spmm(x4, ii2d, io2d, v1d, *, rows_per_tile, in_size):
    mesh = plsc.VectorSubcoreMesh(core_axis_name="c", subcore_axis_name="s")
    out_type = jax.ShapeDtypeStruct((NCHUNK * OUT_SIZE, WCHUNK), jnp.bfloat16)
    rows_out = OUT_SIZE // NSUB
    rows_in = in_size // NSUB

    stages = []
    r = 0
    while r < rows_per_tile:
        n = min(SROWS, rows_per_tile - r)
        assert n % NRING == 0 and n % 8 == 0
        stages.append((r, n))
        r += n

    def body(x_hbm, ii_hbm, io_hbm, v_hbm, out_hbm,
             acc, x_sp, ii_v, io_v, v_v, rb0, rb1, rb2, rb3, gsem, ssem):
        c = lax.axis_index("c")
        s = lax.axis_index("s")
        row0 = s * rows_per_tile
        rbufs = [rb0, rb1, rb2, rb3]

        zero32 = jnp.zeros((2 * LANES,), jnp.bfloat16)

        def mul_block(rb, j):
            @plsc.parallel_loop(0, EBLK, unroll=4)
            def _(e):
                vspl = plsc.load_gather(
                    v_v, [jnp.full((LANES,), j * EBLK + e, jnp.int32)])
                vsplh = plsc.pack(vspl, vspl,
                                  format=plsc.PackFormat.INTERLEAVED)
                for g in range(WCHUNK // (2 * LANES)):
                    sl = pl.ds(g * 2 * LANES, 2 * LANES)
                    rb[e, sl] = rb[e, sl] * vsplh

        for ci in range(NCHUNK // NCORES):
            chunk = c * (NCHUNK // NCORES) + ci

            pltpu.sync_copy(
                x_hbm.at[pl.ds(chunk * in_size + s * rows_in, rows_in)],
                x_sp.at[pl.ds(s * rows_in, rows_in)])

            @pl.loop(0, EBLK)
            def _(r):
                for k in range(WCHUNK // (2 * LANES)):
                    rb0[r, pl.ds(k * 2 * LANES, 2 * LANES)] = zero32

            @pl.loop(0, rows_out // EBLK)
            def _(k):
                pltpu.sync_copy(rb0, acc.at[pl.ds(s * rows_out + k * EBLK, EBLK)])
            plsc.subcore_barrier()

            for srow, slen in stages:
                brow = row0 + srow
                pltpu.sync_copy(ii_hbm.at[pl.ds(brow, slen)],
                                ii_v.at[pl.ds(0, slen)])
                pltpu.sync_copy(io_hbm.at[pl.ds(brow, slen)],
                                io_v.at[pl.ds(0, slen)])
                pltpu.sync_copy(v_hbm.at[pl.ds(brow * EBLK, slen * EBLK)],
                                v_v.at[pl.ds(0, slen * EBLK)])

                for q in range(NRING - 1):
                    pltpu.async_copy(x_sp.at[ii_v.at[q]], rbufs[q], gsem)

                @pl.loop(0, slen // NRING)
                def _(p):
                    for q in range(NRING):
                        j = p * NRING + q
                        rb = rbufs[q]
                        pltpu.make_async_copy(
                            x_sp.at[ii_v.at[j]], rb, gsem).wait()

                        @pl.when(j >= 1)
                        def _():
                            jm = j - 1
                            pltpu.make_async_copy(
                                rbufs[(q + NRING - 1) % NRING],
                                acc.at[io_v.at[jm]], ssem).wait()

                        @pl.when(j + NRING - 1 < slen)
                        def _():
                            jn = j + NRING - 1
                            pltpu.async_copy(
                                x_sp.at[ii_v.at[jn]],
                                rbufs[(q + NRING - 1) % NRING], gsem)

                        mul_block(rb, j)
                        pltpu.async_copy(rb, acc.at[io_v.at[j]], ssem,
                                         add=True)

                pltpu.make_async_copy(
                    rbufs[(slen - 1) % NRING],
                    acc.at[io_v.at[slen - 1]], ssem).wait()

            plsc.subcore_barrier()
            pltpu.sync_copy(
                acc.at[pl.ds(s * rows_out, rows_out)],
                out_hbm.at[pl.ds(chunk * OUT_SIZE + s * rows_out, rows_out)])
            plsc.subcore_barrier()

    cp = pltpu.CompilerParams()
    if "needs_layout_passes" in pltpu.CompilerParams.__dataclass_fields__:
        cp = dataclasses.replace(cp, needs_layout_passes=False)
    if "use_tc_tiling_on_sc" in pltpu.CompilerParams.__dataclass_fields__:
        cp = dataclasses.replace(cp, use_tc_tiling_on_sc=False)
    run = pl.kernel(
        body,
        out_type=out_type,
        mesh=mesh,
        compiler_params=cp,
        scratch_types=[
            pltpu.VMEM_SHARED((OUT_SIZE, WCHUNK), jnp.bfloat16),
            pltpu.VMEM_SHARED((in_size, WCHUNK), jnp.bfloat16),
            pltpu.VMEM((SROWS, EBLK), jnp.int32),
            pltpu.VMEM((SROWS, EBLK), jnp.int32),
            pltpu.VMEM((SROWS * EBLK,), jnp.float32),
            pltpu.VMEM((EBLK, WCHUNK), jnp.bfloat16),
            pltpu.VMEM((EBLK, WCHUNK), jnp.bfloat16),
            pltpu.VMEM((EBLK, WCHUNK), jnp.bfloat16),
            pltpu.VMEM((EBLK, WCHUNK), jnp.bfloat16),
            pltpu.SemaphoreType.DMA,
            pltpu.SemaphoreType.DMA,
        ],
    )
    return run(x4, ii2d, io2d, v1d)


@jax.jit
def kernel(x, v, indices_in, indices_out):
    batch, in_size = x.shape
    nnz = v.shape[0]
    assert batch == NCHUNK * WCHUNK

    per_tile = -(-nnz // (NSUB * EBLK * 8)) * EBLK * 8
    nnz_pad = per_tile * NSUB
    pad = nnz_pad - nnz
    ii = jnp.concatenate([indices_in, jnp.zeros((pad,), jnp.int32)])
    io = jnp.concatenate([indices_out, jnp.zeros((pad,), jnp.int32)])
    vp = jnp.concatenate([v, jnp.zeros((pad,), jnp.float32)])
    ii2d = ii.reshape(nnz_pad // EBLK, EBLK)
    io2d = io.reshape(nnz_pad // EBLK, EBLK)

    x4 = (x.T.reshape(in_size, NCHUNK, WCHUNK)
          .transpose(1, 0, 2).reshape(NCHUNK * in_size, WCHUNK))
    x4 = x4.astype(jnp.bfloat16)

    yt4 = _sc_spmm(x4, ii2d, io2d, vp,
                   rows_per_tile=per_tile // EBLK, in_size=in_size)

    y = (yt4.astype(jnp.float32).reshape(NCHUNK, OUT_SIZE, WCHUNK)
         .transpose(1, 0, 2).reshape(OUT_SIZE, batch).T)
    return y
